# baseline (speedup 1.0000x reference)
import functools

import jax
import jax.numpy as jnp
import numpy as np
from jax import lax
from jax.experimental import pallas as pl
from jax.experimental.pallas import tpu as pltpu
from jax.experimental.pallas import tpu_sc as plsc

G = 100
S = 500
N = G * S
E = 800000
D = 64
K1, K2, K3 = 400, 320, 256

_NC, _NS, _LANES = 2, 16, 16
_NW = _NC * _NS

_C1_CHUNK = 3584
_C1_PER_TILE = 7 * _C1_CHUNK
_E_PAD = _NW * _C1_PER_TILE
_CNT_W = 4 * N + 64
_CNT_TILE = _CNT_W // _NS
_ZB_W = 12512


def _conv1_count_body(src_hbm, dst_hbm, x_hbm, out_hbm,
                      src_v, dst_v, xg_v, flat2_v, ones_v, zb_v, cnt_sp, gsem):
    cid = lax.axis_index("c")
    sid = lax.axis_index("s")
    wid = sid * _NC + cid

    def zero_body(j, _):
        zb_v[pl.ds(j * _LANES, _LANES)] = jnp.zeros((_LANES,), jnp.int32)
        return 0
    lax.fori_loop(0, _ZB_W // _LANES, zero_body, 0, unroll=8)
    pltpu.sync_copy(zb_v.at[pl.ds(0, _CNT_TILE)],
                    cnt_sp.at[pl.ds(sid * _CNT_TILE, _CNT_TILE)])
    for j in range(128 // _LANES):
        ones_v[pl.ds(j * _LANES, _LANES)] = jnp.ones((_LANES,), jnp.int32)
    plsc.subcore_barrier()

    def chunk_body(k, _):
        base = wid * _C1_PER_TILE + k * _C1_CHUNK
        pltpu.sync_copy(src_hbm.at[pl.ds(base, _C1_CHUNK)], src_v)
        pltpu.sync_copy(dst_hbm.at[pl.ds(base, _C1_CHUNK)], dst_v)
        for g in range(28):
            pltpu.async_copy(x_hbm.at[src_v.at[pl.ds(g * 128, 128)]],
                             xg_v.at[pl.ds(g * 128, 128)], gsem)
        for g in range(28):
            pltpu.make_async_copy(x_hbm.at[src_v.at[pl.ds(g * 128, 128)]],
                                  xg_v.at[pl.ds(g * 128, 128)], gsem).wait()
        def vec_body(i, _):
            g = i // 8
            col = (i % 8) * _LANES
            d = dst_v[pl.ds(i * _LANES, _LANES)]
            xs = xg_v[pl.ds(i * _LANES, _LANES)]
            flat2_v[g, pl.ds(col, _LANES)] = d * 4 + xs
            return 0
        lax.fori_loop(0, _C1_CHUNK // _LANES, vec_body, 0, unroll=8)
        for g in range(28):
            pltpu.sync_copy(ones_v, cnt_sp.at[flat2_v.at[g]], add=True)
        return 0

    lax.fori_loop(0, _C1_PER_TILE // _C1_CHUNK, chunk_body, 0)
    plsc.subcore_barrier()
    pltpu.sync_copy(cnt_sp.at[pl.ds(sid * _CNT_TILE, _CNT_TILE)],
                    zb_v.at[pl.ds(0, _CNT_TILE)])
    pltpu.sync_copy(zb_v.at[pl.ds(0, _CNT_TILE)],
                    out_hbm.at[pl.ds(cid * _CNT_W + sid * _CNT_TILE, _CNT_TILE)])


def _conv1_counts(src_pad, dst_pad, xflat):
    mesh = plsc.VectorSubcoreMesh(core_axis_name="c", subcore_axis_name="s",
                                  num_cores=_NC, num_subcores=_NS)
    return pl.kernel(
        _conv1_count_body,
        out_type=jax.ShapeDtypeStruct((_NC * _CNT_W,), jnp.int32),
        mesh=mesh,
        scratch_types=[
            pltpu.VMEM((_C1_CHUNK,), jnp.int32),
            pltpu.VMEM((_C1_CHUNK,), jnp.int32),
            pltpu.VMEM((_C1_CHUNK,), jnp.int32),
            pltpu.VMEM((28, 128), jnp.int32),
            pltpu.VMEM((128,), jnp.int32),
            pltpu.VMEM((_ZB_W,), jnp.int32),
            pltpu.VMEM_SHARED((_CNT_W,), jnp.int32),
            pltpu.SemaphoreType.DMA,
        ],
    )(src_pad, dst_pad, xflat)


def _conv_dense_body(agg_ref, h_ref, wua_ref, wub_ref, wn_ref, h_out, s_out):
    hp = jnp.maximum(
        jnp.dot(agg_ref[...], wua_ref[...], preferred_element_type=jnp.float32)
        + jnp.dot(h_ref[...], wub_ref[...], preferred_element_type=jnp.float32),
        0.0)
    h_out[...] = hp
    s_out[...] = jnp.tanh(jnp.dot(hp, wn_ref[...],
                                  preferred_element_type=jnp.float32))


def _conv_dense(agg, h, WuA_T, WuB_T, wn, bm=1000):
    m = agg.shape[0]
    grid = (pl.cdiv(m, bm),)
    return pl.pallas_call(
        _conv_dense_body,
        grid=grid,
        in_specs=[
            pl.BlockSpec((bm, D), lambda i: (i, 0)),
            pl.BlockSpec((bm, D), lambda i: (i, 0)),
            pl.BlockSpec((D, D), lambda i: (0, 0)),
            pl.BlockSpec((D, D), lambda i: (0, 0)),
            pl.BlockSpec((D, 1), lambda i: (0, 0)),
        ],
        out_specs=[
            pl.BlockSpec((bm, D), lambda i: (i, 0)),
            pl.BlockSpec((bm, 1), lambda i: (i, 0)),
        ],
        out_shape=[
            jax.ShapeDtypeStruct((m, D), jnp.float32),
            jax.ShapeDtypeStruct((m, 1), jnp.float32),
        ],
    )(agg, h, WuA_T, WuB_T, wn)


def _conv1_dense_body(cA_ref, cB_ref, x_ref, t4_ref, e4_ref, wua_ref, wn_ref,
                      h_ref, s_ref):
    x = x_ref[...]
    agg = jnp.full(h_ref.shape, -3e38, jnp.float32)
    e4sel = jnp.zeros(h_ref.shape, jnp.float32)
    for c in range(4):
        m = (cA_ref[:, c:c + 1] + cB_ref[:, c:c + 1] > 0) | (x == c)
        agg = jnp.maximum(agg, jnp.where(m, t4_ref[c:c + 1, :], -3e38))
        e4sel = e4sel + jnp.where(x == c, e4_ref[c:c + 1, :], 0.0)
    h1 = jnp.maximum(
        jnp.dot(agg, wua_ref[...], preferred_element_type=jnp.float32) + e4sel,
        0.0)
    h_ref[...] = h1
    s_ref[...] = jnp.tanh(jnp.dot(h1, wn_ref[...],
                                  preferred_element_type=jnp.float32))


def _conv1_dense(cntA, cntB, x, T4, E4, WuA_T, wn, bm=1000):
    grid = (pl.cdiv(N, bm),)
    return pl.pallas_call(
        _conv1_dense_body,
        grid=grid,
        in_specs=[
            pl.BlockSpec((bm, 4), lambda i: (i, 0)),
            pl.BlockSpec((bm, 4), lambda i: (i, 0)),
            pl.BlockSpec((bm, 1), lambda i: (i, 0)),
            pl.BlockSpec((4, D), lambda i: (0, 0)),
            pl.BlockSpec((4, D), lambda i: (0, 0)),
            pl.BlockSpec((D, D), lambda i: (0, 0)),
            pl.BlockSpec((D, 1), lambda i: (0, 0)),
        ],
        out_specs=[
            pl.BlockSpec((bm, D), lambda i: (i, 0)),
            pl.BlockSpec((bm, 1), lambda i: (i, 0)),
        ],
        out_shape=[
            jax.ShapeDtypeStruct((N, D), jnp.float32),
            jax.ShapeDtypeStruct((N, 1), jnp.float32),
        ],
    )(cntA, cntB, x, T4, E4, WuA_T, wn)


def _linear_body(h_ref, w_ref, b_ref, o_ref, *, act):
    y = jnp.dot(h_ref[...], w_ref[...], preferred_element_type=jnp.float32)
    y = y + b_ref[...]
    if act == "relu":
        y = jnp.maximum(y, 0.0)
    elif act == "sigmoid":
        y = jax.nn.sigmoid(y)
    o_ref[...] = y


def _linear(h, Wt, b, act="relu", bm=512):
    M, Din = h.shape
    Dout = Wt.shape[1]
    grid = (pl.cdiv(M, bm),)
    return pl.pallas_call(
        functools.partial(_linear_body, act=act),
        grid=grid,
        in_specs=[
            pl.BlockSpec((bm, Din), lambda i: (i, 0)),
            pl.BlockSpec((Din, Dout), lambda i: (0, 0)),
            pl.BlockSpec((1, Dout), lambda i: (0, 0)),
        ],
        out_specs=pl.BlockSpec((bm, Dout), lambda i: (i, 0)),
        out_shape=jax.ShapeDtypeStruct((M, Dout), jnp.float32),
    )(h, Wt, b.reshape(1, Dout))


def _segment_max_jax(t, ns, nd, m):
    t_pad = jnp.concatenate([t, jnp.zeros((1, D), jnp.float32)], axis=0)
    agg = jax.ops.segment_max(t_pad[ns], nd, num_segments=m + 1)[:m]
    return jnp.maximum(agg, t)


def _topk_pool(h, src, dst, w, n_graphs, s_per, k, score=None):
    if score is None:
        score = jnp.tanh((h @ w) / jnp.linalg.norm(w))
    _, idx = jax.lax.top_k(score.reshape(n_graphs, s_per), k)
    perm = (idx + (jnp.arange(n_graphs) * s_per)[:, None]).reshape(-1)
    new_h = h[perm] * score[perm][:, None]
    M_new = n_graphs * k
    mapping = jnp.full((n_graphs * s_per + 1,), M_new, dtype=src.dtype)
    mapping = mapping.at[perm].set(jnp.arange(M_new, dtype=src.dtype))
    ns = mapping[src]
    nd = mapping[dst]
    valid = (ns < M_new) & (nd < M_new)
    ns = jnp.where(valid, ns, M_new)
    nd = jnp.where(valid, nd, M_new)
    return new_h, ns, nd


def _readout(h, n_graphs, k):
    hb = h.reshape(n_graphs, k, -1)
    return jnp.concatenate([hb.max(axis=1), hb.mean(axis=1)], axis=1)


def kernel(params, x, edge_index, batch):
    src, dst = edge_index[0], edge_index[1]
    emb = params["emb"]
    Wl, bl, Wu = params["conv1"]

    xflat = x[:, 0]
    src_pad = jnp.concatenate([src, jnp.zeros((_E_PAD - E,), src.dtype)])
    dst_pad = jnp.concatenate([dst, jnp.full((_E_PAD - E,), N, dst.dtype)])
    cnt2 = _conv1_counts(src_pad, dst_pad, xflat)
    cntA = cnt2[:4 * N].reshape(N, 4)
    cntB = cnt2[_CNT_W:_CNT_W + 4 * N].reshape(N, 4)
    T4 = _linear(emb, Wl.T, bl, act="relu", bm=8)
    zero64 = jnp.zeros((D,), jnp.float32)
    E4 = _linear(emb, Wu[:, D:].T, zero64, act="none", bm=8)
    w1 = params["pool1"]
    wn1 = (w1 / jnp.linalg.norm(w1)).reshape(D, 1)
    h, score1 = _conv1_dense(cntA, cntB, x, T4, E4, Wu[:, :D].T, wn1)
    h, src, dst = _topk_pool(h, src, dst, params["pool1"], G, S, K1,
                             score=score1[:, 0])
    x1 = _readout(h, G, K1)

    Wl, bl, Wu = params["conv2"]
    t = _linear(h, Wl.T, bl, act="relu")
    agg = _segment_max_jax(t, src, dst, G * K1)
    w2 = params["pool2"]
    wn2 = (w2 / jnp.linalg.norm(w2)).reshape(D, 1)
    h, score2 = _conv_dense(agg, h, Wu[:, :D].T, Wu[:, D:].T, wn2)
    h, src, dst = _topk_pool(h, src, dst, w2, G, K1, K2, score=score2[:, 0])
    x2 = _readout(h, G, K2)

    Wl, bl, Wu = params["conv3"]
    t = _linear(h, Wl.T, bl, act="relu")
    agg = _segment_max_jax(t, src, dst, G * K2)
    w3 = params["pool3"]
    wn3 = (w3 / jnp.linalg.norm(w3)).reshape(D, 1)
    h, score3 = _conv_dense(agg, h, Wu[:, :D].T, Wu[:, D:].T, wn3)
    h, src, dst = _topk_pool(h, src, dst, w3, G, K2, K3, score=score3[:, 0])
    x3 = _readout(h, G, K3)
    z = x1 + x2 + x3
    W, b = params["lin1"]
    z = _linear(z, W.T, b, act="relu", bm=128)
    W, b = params["lin2"]
    z = _linear(z, W.T, b, act="relu", bm=128)
    W, b = params["lin3"]
    z = _linear(z, W.T, b, act="sigmoid", bm=128)
    batch_out = jnp.repeat(jnp.arange(G), K3)
    return z, batch_out
